# baseline (speedup 1.0000x reference)
import functools

import jax
import jax.numpy as jnp
from jax.experimental import pallas as pl
from jax.experimental.pallas import tpu as pltpu


def _gather_scale_body(fidx_ref, yh_ref, x_ref, o_ref):
    b = pl.program_id(0)
    k = pl.program_id(1)
    scale = yh_ref[b, k]
    o_ref[...] = x_ref[...] * scale


def _gather_scale(x3, fidx, y_half, half):
    B = y_half.shape[0]
    grid_spec = pltpu.PrefetchScalarGridSpec(
        num_scalar_prefetch=2,
        grid=(B, half),
        in_specs=[
            pl.BlockSpec((1, 392, 128), lambda b, k, fidx, yh: (fidx[b, k], 0, 0)),
        ],
        out_specs=pl.BlockSpec((1, 392, 128), lambda b, k, fidx, yh: (b * half + k, 0, 0)),
    )
    out = pl.pallas_call(
        _gather_scale_body,
        grid_spec=grid_spec,
        out_shape=jax.ShapeDtypeStruct((B * half, 392, 128), jnp.float32),
        compiler_params=pltpu.CompilerParams(
            dimension_semantics=("arbitrary", "arbitrary"),
        ),
    )(fidx, y_half, x3)
    return out


def kernel(x, w1, w2):
    b, c, h, w = x.shape
    half = c // 2
    y = jnp.mean(x, axis=(2, 3))
    y = y @ w1.T
    y = jax.nn.relu(y)
    y = y @ w2.T
    y = jax.nn.sigmoid(y)
    order = jnp.argsort(-y, axis=1)
    idx = order[:, :half]
    y_half = jnp.take_along_axis(y, idx, axis=1)

    fidx = (idx + c * jnp.arange(b, dtype=idx.dtype)[:, None]).astype(jnp.int32)
    x3 = x.reshape(b * c, 392, 128)
    out = _gather_scale(x3, fidx, y_half, half)
    return out.reshape(b, half, h, w)


# speedup vs baseline: 1.3091x; 1.3091x over previous
import functools

import jax
import jax.numpy as jnp
from jax import lax
from jax.experimental import pallas as pl
from jax.experimental.pallas import tpu as pltpu
from jax.experimental.pallas import tpu_sc as plsc

_B, _C = 4, 384
_HALF = _C // 2
_NVREG = _C // 16
_ROT = _C * 16


def _select_kernel(yrot_hbm, sidx_hbm, idx_hbm, yh_hbm, yv, sv, li, lg, mi, mg, shi, shg):
    cid = lax.axis_index("c")
    sid = lax.axis_index("s")
    wid = cid * 16 + sid
    bb = wid // 8
    row = bb % 2
    chunk = wid % 8

    pltpu.sync_copy(yrot_hbm.at[bb], yv)
    pltpu.sync_copy(sidx_hbm, sv)

    iota = lax.iota(jnp.int32, 16)
    ones = jnp.ones((16,), jnp.int32)
    zi = jnp.zeros((16,), jnp.int32)
    zf = jnp.zeros((16,), jnp.float32)

    t_val = [yv[pl.ds((chunk * 3 + i) * 16, 16)] for i in range(3)]
    t_idx = [iota + (chunk * 48 + i * 16) for i in range(3)]

    def body(it, cnts):
        off = it * 16
        srot = yv[pl.ds(off, 16)]
        si = sv[pl.ds(off, 16)]
        out = []
        for i in range(3):
            beats = (srot > t_val[i]) | ((srot == t_val[i]) & (si < t_idx[i]))
            out.append(cnts[i] + jnp.where(beats, ones, zi))
        return tuple(out)

    ranks = lax.fori_loop(0, _ROT // 16, body, (zi, zi, zi))

    for q in range(_NVREG):
        li[pl.ds(q * 16, 16)] = zi
        lg[pl.ds(q * 16, 16)] = zf

    for i in range(3):
        plsc.store_scatter(li, [ranks[i]], t_idx[i])
        plsc.store_scatter(lg, [ranks[i]], t_val[i])

    pltpu.sync_copy(li, shi.at[pl.ds(sid * _C, _C)])
    pltpu.sync_copy(lg, shg.at[pl.ds(sid * _C, _C)])

    plsc.subcore_barrier()

    @pl.when(chunk == 0)
    def _writeback():
        for q in range(_NVREG):
            li[pl.ds(q * 16, 16)] = zi
            lg[pl.ds(q * 16, 16)] = zf
        for t in range(8):
            slot = (row * 8 + t) * _C
            pltpu.sync_copy(shi.at[pl.ds(slot, _C)], mi)
            pltpu.sync_copy(shg.at[pl.ds(slot, _C)], mg)
            for q in range(_NVREG):
                sl = pl.ds(q * 16, 16)
                li[sl] = li[sl] + mi[sl]
                lg[sl] = lg[sl] + mg[sl]
        pltpu.sync_copy(li.at[pl.ds(0, _HALF)], idx_hbm.at[bb])
        pltpu.sync_copy(lg.at[pl.ds(0, _HALF)], yh_hbm.at[bb])


@functools.lru_cache(maxsize=1)
def _select():
    return functools.partial(
        pl.kernel,
        out_type=(
            jax.ShapeDtypeStruct((_B, _HALF), jnp.int32),
            jax.ShapeDtypeStruct((_B, _HALF), jnp.float32),
        ),
        mesh=plsc.VectorSubcoreMesh(
            core_axis_name="c", subcore_axis_name="s", num_cores=2, num_subcores=16
        ),
        compiler_params=pltpu.CompilerParams(
            needs_layout_passes=False, use_tc_tiling_on_sc=False
        ),
        scratch_types=[
            pltpu.VMEM((_ROT,), jnp.float32),
            pltpu.VMEM((_ROT,), jnp.int32),
            pltpu.VMEM((_C,), jnp.int32),
            pltpu.VMEM((_C,), jnp.float32),
            pltpu.VMEM((_C,), jnp.int32),
            pltpu.VMEM((_C,), jnp.float32),
            pltpu.VMEM_SHARED((16 * _C,), jnp.int32),
            pltpu.VMEM_SHARED((16 * _C,), jnp.float32),
        ],
    )(_select_kernel)


_NP = 8


def _gather_scale_body(fidx_ref, yh_ref, *refs):
    i = pl.program_id(0)
    xs, o_ref = refs[:_NP], refs[_NP]
    for j in range(_NP):
        o_ref[j] = xs[j][0] * yh_ref[i * _NP + j]


def _gather_scale(x3, fidx, y_half):
    n = fidx.shape[0]

    def mk(j):
        return lambda i, fidx, yh: (fidx[i * _NP + j], 0, 0)

    grid_spec = pltpu.PrefetchScalarGridSpec(
        num_scalar_prefetch=2,
        grid=(n // _NP,),
        in_specs=[pl.BlockSpec((1, 392, 128), mk(j)) for j in range(_NP)],
        out_specs=pl.BlockSpec((_NP, 392, 128), lambda i, fidx, yh: (i, 0, 0)),
    )
    return pl.pallas_call(
        _gather_scale_body,
        grid_spec=grid_spec,
        out_shape=jax.ShapeDtypeStruct((n, 392, 128), jnp.float32),
        compiler_params=pltpu.CompilerParams(
            dimension_semantics=("arbitrary",),
        ),
    )(fidx, y_half, *([x3] * _NP))


def kernel(x, w1, w2):
    b, c, h, w = x.shape
    half = c // 2
    y = jnp.mean(x, axis=(2, 3))
    y = y @ w1.T
    y = jax.nn.relu(y)
    y = y @ w2.T
    y = jax.nn.sigmoid(y)
    y = lax.optimization_barrier(y)

    yv4 = y.reshape(b, _NVREG, 16)
    yrot = jnp.concatenate(
        [jnp.roll(yv4, -r, axis=2).reshape(b, 1, c) for r in range(16)], axis=1
    ).reshape(b, 16 * c)
    l16 = jnp.arange(16, dtype=jnp.int32)
    sidx = (
        jnp.arange(_NVREG, dtype=jnp.int32)[None, :, None] * 16
        + (l16[None, None, :] + l16[:, None, None]) % 16
    ).reshape(16 * c)

    idx, y_half = _select()(yrot, sidx)

    fidx = (idx + c * jnp.arange(b, dtype=jnp.int32)[:, None]).astype(jnp.int32)
    x3 = x.reshape(b * c, 392, 128)
    out = _gather_scale(x3, fidx.reshape(b * half), y_half.reshape(b * half))
    return out.reshape(b, half, h, w)


# speedup vs baseline: 2.4954x; 1.9063x over previous
import functools

import jax
import jax.numpy as jnp
from jax import lax
from jax.experimental import pallas as pl
from jax.experimental.pallas import tpu as pltpu
from jax.experimental.pallas import tpu_sc as plsc

_B, _C = 4, 384
_HALF = _C // 2
_NVREG = _C // 16
_ROT = _C * 16


def _select_kernel(yrot_hbm, sidx_hbm, idx_hbm, yh_hbm, yv, sv, li, lg, mi, mg, shi, shg):
    cid = lax.axis_index("c")
    sid = lax.axis_index("s")
    wid = cid * 16 + sid
    bb = wid // 8
    row = bb % 2
    chunk = wid % 8

    pltpu.sync_copy(yrot_hbm.at[bb], yv)
    pltpu.sync_copy(sidx_hbm, sv)

    iota = lax.iota(jnp.int32, 16)
    ones = jnp.ones((16,), jnp.int32)
    zi = jnp.zeros((16,), jnp.int32)
    zf = jnp.zeros((16,), jnp.float32)

    t_val = [yv[pl.ds((chunk * 3 + i) * 16, 16)] for i in range(3)]
    t_idx = [iota + (chunk * 48 + i * 16) for i in range(3)]

    def body(it, cnts):
        off = it * 16
        srot = yv[pl.ds(off, 16)]
        si = sv[pl.ds(off, 16)]
        out = []
        for i in range(3):
            beats = (srot > t_val[i]) | ((srot == t_val[i]) & (si < t_idx[i]))
            out.append(cnts[i] + jnp.where(beats, ones, zi))
        return tuple(out)

    ranks = lax.fori_loop(0, _ROT // 16, body, (zi, zi, zi))

    for q in range(_NVREG):
        li[pl.ds(q * 16, 16)] = zi
        lg[pl.ds(q * 16, 16)] = zf

    for i in range(3):
        plsc.store_scatter(li, [ranks[i]], t_idx[i])
        plsc.store_scatter(lg, [ranks[i]], t_val[i])

    pltpu.sync_copy(li, shi.at[pl.ds(sid * _C, _C)])
    pltpu.sync_copy(lg, shg.at[pl.ds(sid * _C, _C)])

    plsc.subcore_barrier()

    @pl.when(chunk == 0)
    def _writeback():
        for q in range(_NVREG):
            li[pl.ds(q * 16, 16)] = zi
            lg[pl.ds(q * 16, 16)] = zf
        for t in range(8):
            slot = (row * 8 + t) * _C
            pltpu.sync_copy(shi.at[pl.ds(slot, _C)], mi)
            pltpu.sync_copy(shg.at[pl.ds(slot, _C)], mg)
            for q in range(_NVREG):
                sl = pl.ds(q * 16, 16)
                li[sl] = li[sl] + mi[sl]
                lg[sl] = lg[sl] + mg[sl]
        pltpu.sync_copy(li.at[pl.ds(0, _HALF)], idx_hbm.at[bb])
        pltpu.sync_copy(lg.at[pl.ds(0, _HALF)], yh_hbm.at[bb])


@functools.lru_cache(maxsize=1)
def _select():
    return functools.partial(
        pl.kernel,
        out_type=(
            jax.ShapeDtypeStruct((_B, _HALF), jnp.int32),
            jax.ShapeDtypeStruct((_B, _HALF), jnp.float32),
        ),
        mesh=plsc.VectorSubcoreMesh(
            core_axis_name="c", subcore_axis_name="s", num_cores=2, num_subcores=16
        ),
        compiler_params=pltpu.CompilerParams(
            needs_layout_passes=False, use_tc_tiling_on_sc=False
        ),
        scratch_types=[
            pltpu.VMEM((_ROT,), jnp.float32),
            pltpu.VMEM((_ROT,), jnp.int32),
            pltpu.VMEM((_C,), jnp.int32),
            pltpu.VMEM((_C,), jnp.float32),
            pltpu.VMEM((_C,), jnp.int32),
            pltpu.VMEM((_C,), jnp.float32),
            pltpu.VMEM_SHARED((16 * _C,), jnp.int32),
            pltpu.VMEM_SHARED((16 * _C,), jnp.float32),
        ],
    )(_select_kernel)


_NP = 8


def _gather_scale_body(idx_ref, yh_ref, *refs):
    b = pl.program_id(0)
    k = pl.program_id(1)
    xs, o_ref = refs[:_NP], refs[_NP]
    for j in range(_NP):
        o_ref[0, j] = xs[j][0, 0] * yh_ref[b, k * _NP + j]


def _gather_scale(x, idx, y_half):
    B, C, H, W = x.shape

    def mk(j):
        return lambda b, k, idx, yh: (b, idx[b, k * _NP + j], 0, 0)

    grid_spec = pltpu.PrefetchScalarGridSpec(
        num_scalar_prefetch=2,
        grid=(B, _HALF // _NP),
        in_specs=[pl.BlockSpec((1, 1, H, W), mk(j)) for j in range(_NP)],
        out_specs=pl.BlockSpec(
            (1, _NP, H, W), lambda b, k, idx, yh: (b, k, 0, 0)
        ),
    )
    return pl.pallas_call(
        _gather_scale_body,
        grid_spec=grid_spec,
        out_shape=jax.ShapeDtypeStruct((B, _HALF, H, W), jnp.float32),
        compiler_params=pltpu.CompilerParams(
            dimension_semantics=("arbitrary", "arbitrary"),
        ),
    )(idx, y_half, *([x] * _NP))


def kernel(x, w1, w2):
    b, c, h, w = x.shape
    half = c // 2
    y = jnp.mean(x, axis=(2, 3))
    y = y @ w1.T
    y = jax.nn.relu(y)
    y = y @ w2.T
    y = jax.nn.sigmoid(y)
    y = lax.optimization_barrier(y)

    yv4 = y.reshape(b, _NVREG, 16)
    yrot = jnp.concatenate(
        [jnp.roll(yv4, -r, axis=2).reshape(b, 1, c) for r in range(16)], axis=1
    ).reshape(b, 16 * c)
    l16 = jnp.arange(16, dtype=jnp.int32)
    sidx = (
        jnp.arange(_NVREG, dtype=jnp.int32)[None, :, None] * 16
        + (l16[None, None, :] + l16[:, None, None]) % 16
    ).reshape(16 * c)

    idx, y_half = _select()(yrot, sidx)

    return _gather_scale(x, idx, y_half)


# speedup vs baseline: 2.5603x; 1.0260x over previous
import functools

import jax
import jax.numpy as jnp
from jax import lax
from jax.experimental import pallas as pl
from jax.experimental.pallas import tpu as pltpu
from jax.experimental.pallas import tpu_sc as plsc

_B, _C = 4, 384
_HALF = _C // 2
_NVREG = _C // 16
_ROT = _C * 16


def _select_kernel(yrot_hbm, sidx_hbm, idx_hbm, yh_hbm, yv, sv, li, lg, mi, mg, shi, shg):
    cid = lax.axis_index("c")
    sid = lax.axis_index("s")
    wid = cid * 16 + sid
    bb = wid // 8
    row = bb % 2
    chunk = wid % 8

    pltpu.sync_copy(yrot_hbm.at[bb], yv)
    pltpu.sync_copy(sidx_hbm, sv)

    iota = lax.iota(jnp.int32, 16)
    ones = jnp.ones((16,), jnp.int32)
    zi = jnp.zeros((16,), jnp.int32)
    zf = jnp.zeros((16,), jnp.float32)

    t_val = [yv[pl.ds((chunk * 3 + i) * 16, 16)] for i in range(3)]
    t_idx = [iota + (chunk * 48 + i * 16) for i in range(3)]

    def body(it, cnts):
        off = it * 16
        srot = yv[pl.ds(off, 16)]
        si = sv[pl.ds(off, 16)]
        out = []
        for i in range(3):
            beats = (srot > t_val[i]) | ((srot == t_val[i]) & (si < t_idx[i]))
            out.append(cnts[i] + jnp.where(beats, ones, zi))
        return tuple(out)

    ranks = lax.fori_loop(0, _ROT // 16, body, (zi, zi, zi))

    for q in range(_NVREG):
        li[pl.ds(q * 16, 16)] = zi
        lg[pl.ds(q * 16, 16)] = zf

    for i in range(3):
        plsc.store_scatter(li, [ranks[i]], t_idx[i])
        plsc.store_scatter(lg, [ranks[i]], t_val[i])

    pltpu.sync_copy(li, shi.at[pl.ds(sid * _C, _C)])
    pltpu.sync_copy(lg, shg.at[pl.ds(sid * _C, _C)])

    plsc.subcore_barrier()

    @pl.when(chunk == 0)
    def _writeback():
        for q in range(_NVREG):
            li[pl.ds(q * 16, 16)] = zi
            lg[pl.ds(q * 16, 16)] = zf
        for t in range(8):
            slot = (row * 8 + t) * _C
            pltpu.sync_copy(shi.at[pl.ds(slot, _C)], mi)
            pltpu.sync_copy(shg.at[pl.ds(slot, _C)], mg)
            for q in range(_NVREG):
                sl = pl.ds(q * 16, 16)
                li[sl] = li[sl] + mi[sl]
                lg[sl] = lg[sl] + mg[sl]
        pltpu.sync_copy(li.at[pl.ds(0, _HALF)], idx_hbm.at[bb])
        pltpu.sync_copy(lg.at[pl.ds(0, _HALF)], yh_hbm.at[bb])


@functools.lru_cache(maxsize=1)
def _select():
    return functools.partial(
        pl.kernel,
        out_type=(
            jax.ShapeDtypeStruct((_B, _HALF), jnp.int32),
            jax.ShapeDtypeStruct((_B, _HALF), jnp.float32),
        ),
        mesh=plsc.VectorSubcoreMesh(
            core_axis_name="c", subcore_axis_name="s", num_cores=2, num_subcores=16
        ),
        compiler_params=pltpu.CompilerParams(
            needs_layout_passes=False, use_tc_tiling_on_sc=False
        ),
        scratch_types=[
            pltpu.VMEM((_ROT,), jnp.float32),
            pltpu.VMEM((_ROT,), jnp.int32),
            pltpu.VMEM((_C,), jnp.int32),
            pltpu.VMEM((_C,), jnp.float32),
            pltpu.VMEM((_C,), jnp.int32),
            pltpu.VMEM((_C,), jnp.float32),
            pltpu.VMEM_SHARED((16 * _C,), jnp.int32),
            pltpu.VMEM_SHARED((16 * _C,), jnp.float32),
        ],
    )(_select_kernel)


_NP = 16


def _gather_scale_body(idx_ref, yh_ref, *refs):
    b = pl.program_id(0)
    k = pl.program_id(1)
    xs, o_ref = refs[:_NP], refs[_NP]
    for j in range(_NP):
        o_ref[0, j] = xs[j][0, 0] * yh_ref[b, k * _NP + j]


def _gather_scale(x, idx, y_half):
    B, C, H, W = x.shape

    def mk(j):
        return lambda b, k, idx, yh: (b, idx[b, k * _NP + j], 0, 0)

    grid_spec = pltpu.PrefetchScalarGridSpec(
        num_scalar_prefetch=2,
        grid=(B, _HALF // _NP),
        in_specs=[pl.BlockSpec((1, 1, H, W), mk(j)) for j in range(_NP)],
        out_specs=pl.BlockSpec(
            (1, _NP, H, W), lambda b, k, idx, yh: (b, k, 0, 0)
        ),
    )
    return pl.pallas_call(
        _gather_scale_body,
        grid_spec=grid_spec,
        out_shape=jax.ShapeDtypeStruct((B, _HALF, H, W), jnp.float32),
        compiler_params=pltpu.CompilerParams(
            dimension_semantics=("arbitrary", "arbitrary"),
        ),
    )(idx, y_half, *([x] * _NP))


def kernel(x, w1, w2):
    b, c, h, w = x.shape
    half = c // 2
    y = jnp.mean(x, axis=(2, 3))
    y = y @ w1.T
    y = jax.nn.relu(y)
    y = y @ w2.T
    y = jax.nn.sigmoid(y)
    y = lax.optimization_barrier(y)

    yv4 = y.reshape(b, _NVREG, 16)
    yrot = jnp.concatenate(
        [jnp.roll(yv4, -r, axis=2).reshape(b, 1, c) for r in range(16)], axis=1
    ).reshape(b, 16 * c)
    l16 = jnp.arange(16, dtype=jnp.int32)
    sidx = (
        jnp.arange(_NVREG, dtype=jnp.int32)[None, :, None] * 16
        + (l16[None, None, :] + l16[:, None, None]) % 16
    ).reshape(16 * c)

    idx, y_half = _select()(yrot, sidx)

    return _gather_scale(x, idx, y_half)


# speedup vs baseline: 2.5688x; 1.0033x over previous
import functools

import jax
import jax.numpy as jnp
from jax import lax
from jax.experimental import pallas as pl
from jax.experimental.pallas import tpu as pltpu
from jax.experimental.pallas import tpu_sc as plsc

_B, _C = 4, 384
_HALF = _C // 2
_NVREG = _C // 16
_ROT = _C * 16


def _select_kernel(yrot_hbm, sidx_hbm, idx_hbm, yh_hbm, yv, sv, li, lg, mi, mg, shi, shg):
    cid = lax.axis_index("c")
    sid = lax.axis_index("s")
    wid = cid * 16 + sid
    bb = wid // 8
    row = bb % 2
    chunk = wid % 8

    pltpu.sync_copy(yrot_hbm.at[bb], yv)
    pltpu.sync_copy(sidx_hbm, sv)

    iota = lax.iota(jnp.int32, 16)
    ones = jnp.ones((16,), jnp.int32)
    zi = jnp.zeros((16,), jnp.int32)
    zf = jnp.zeros((16,), jnp.float32)

    t_val = [yv[pl.ds((chunk * 3 + i) * 16, 16)] for i in range(3)]
    t_idx = [iota + (chunk * 48 + i * 16) for i in range(3)]

    def body(it, cnts):
        off = it * 16
        srot = yv[pl.ds(off, 16)]
        si = sv[pl.ds(off, 16)]
        out = []
        for i in range(3):
            beats = (srot > t_val[i]) | ((srot == t_val[i]) & (si < t_idx[i]))
            out.append(cnts[i] + jnp.where(beats, ones, zi))
        return tuple(out)

    ranks = lax.fori_loop(0, _ROT // 16, body, (zi, zi, zi))

    for q in range(_NVREG):
        li[pl.ds(q * 16, 16)] = zi
        lg[pl.ds(q * 16, 16)] = zf

    for i in range(3):
        plsc.store_scatter(li, [ranks[i]], t_idx[i])
        plsc.store_scatter(lg, [ranks[i]], t_val[i])

    pltpu.sync_copy(li, shi.at[pl.ds(sid * _C, _C)])
    pltpu.sync_copy(lg, shg.at[pl.ds(sid * _C, _C)])

    plsc.subcore_barrier()

    @pl.when(chunk == 0)
    def _writeback():
        for q in range(_NVREG):
            li[pl.ds(q * 16, 16)] = zi
            lg[pl.ds(q * 16, 16)] = zf
        for t in range(8):
            slot = (row * 8 + t) * _C
            pltpu.sync_copy(shi.at[pl.ds(slot, _C)], mi)
            pltpu.sync_copy(shg.at[pl.ds(slot, _C)], mg)
            for q in range(_NVREG):
                sl = pl.ds(q * 16, 16)
                li[sl] = li[sl] + mi[sl]
                lg[sl] = lg[sl] + mg[sl]
        pltpu.sync_copy(li.at[pl.ds(0, _HALF)], idx_hbm.at[bb])
        pltpu.sync_copy(lg.at[pl.ds(0, _HALF)], yh_hbm.at[bb])


@functools.lru_cache(maxsize=1)
def _select():
    return functools.partial(
        pl.kernel,
        out_type=(
            jax.ShapeDtypeStruct((_B, _HALF), jnp.int32),
            jax.ShapeDtypeStruct((_B, _HALF), jnp.float32),
        ),
        mesh=plsc.VectorSubcoreMesh(
            core_axis_name="c", subcore_axis_name="s", num_cores=2, num_subcores=16
        ),
        compiler_params=pltpu.CompilerParams(
            needs_layout_passes=False, use_tc_tiling_on_sc=False
        ),
        scratch_types=[
            pltpu.VMEM((_ROT,), jnp.float32),
            pltpu.VMEM((_ROT,), jnp.int32),
            pltpu.VMEM((_C,), jnp.int32),
            pltpu.VMEM((_C,), jnp.float32),
            pltpu.VMEM((_C,), jnp.int32),
            pltpu.VMEM((_C,), jnp.float32),
            pltpu.VMEM_SHARED((16 * _C,), jnp.int32),
            pltpu.VMEM_SHARED((16 * _C,), jnp.float32),
        ],
    )(_select_kernel)


_NP = 24


def _gather_scale_body(idx_ref, yh_ref, *refs):
    b = pl.program_id(0)
    k = pl.program_id(1)
    xs, o_ref = refs[:_NP], refs[_NP]
    for j in range(_NP):
        o_ref[0, j] = xs[j][0, 0] * yh_ref[b, k * _NP + j]


def _gather_scale(x, idx, y_half):
    B, C, H, W = x.shape

    def mk(j):
        return lambda b, k, idx, yh: (b, idx[b, k * _NP + j], 0, 0)

    grid_spec = pltpu.PrefetchScalarGridSpec(
        num_scalar_prefetch=2,
        grid=(B, _HALF // _NP),
        in_specs=[pl.BlockSpec((1, 1, H, W), mk(j)) for j in range(_NP)],
        out_specs=pl.BlockSpec(
            (1, _NP, H, W), lambda b, k, idx, yh: (b, k, 0, 0)
        ),
    )
    return pl.pallas_call(
        _gather_scale_body,
        grid_spec=grid_spec,
        out_shape=jax.ShapeDtypeStruct((B, _HALF, H, W), jnp.float32),
        compiler_params=pltpu.CompilerParams(
            dimension_semantics=("arbitrary", "arbitrary"),
        ),
    )(idx, y_half, *([x] * _NP))


def kernel(x, w1, w2):
    b, c, h, w = x.shape
    half = c // 2
    y = jnp.mean(x, axis=(2, 3))
    y = y @ w1.T
    y = jax.nn.relu(y)
    y = y @ w2.T
    y = jax.nn.sigmoid(y)
    y = lax.optimization_barrier(y)

    yv4 = y.reshape(b, _NVREG, 16)
    yrot = jnp.concatenate(
        [jnp.roll(yv4, -r, axis=2).reshape(b, 1, c) for r in range(16)], axis=1
    ).reshape(b, 16 * c)
    l16 = jnp.arange(16, dtype=jnp.int32)
    sidx = (
        jnp.arange(_NVREG, dtype=jnp.int32)[None, :, None] * 16
        + (l16[None, None, :] + l16[:, None, None]) % 16
    ).reshape(16 * c)

    idx, y_half = _select()(yrot, sidx)

    return _gather_scale(x, idx, y_half)


# speedup vs baseline: 2.5693x; 1.0002x over previous
import functools

import jax
import jax.numpy as jnp
from jax import lax
from jax.experimental import pallas as pl
from jax.experimental.pallas import tpu as pltpu
from jax.experimental.pallas import tpu_sc as plsc

_B, _C = 4, 384
_HALF = _C // 2
_NVREG = _C // 16
_ROT = _C * 16


def _select_kernel(yrot_hbm, sidx_hbm, idx_hbm, yh_hbm, yv, sv, li, lg, mi, mg, shi, shg):
    cid = lax.axis_index("c")
    sid = lax.axis_index("s")
    wid = cid * 16 + sid
    bb = wid // 8
    row = bb % 2
    chunk = wid % 8

    pltpu.sync_copy(yrot_hbm.at[bb], yv)
    pltpu.sync_copy(sidx_hbm, sv)

    iota = lax.iota(jnp.int32, 16)
    ones = jnp.ones((16,), jnp.int32)
    zi = jnp.zeros((16,), jnp.int32)
    zf = jnp.zeros((16,), jnp.float32)

    t_val = [yv[pl.ds((chunk * 3 + i) * 16, 16)] for i in range(3)]
    t_idx = [iota + (chunk * 48 + i * 16) for i in range(3)]

    def body(it, cnts):
        off = it * 16
        srot = yv[pl.ds(off, 16)]
        si = sv[pl.ds(off, 16)]
        out = []
        for i in range(3):
            beats = (srot > t_val[i]) | ((srot == t_val[i]) & (si < t_idx[i]))
            out.append(cnts[i] + jnp.where(beats, ones, zi))
        return tuple(out)

    ranks = lax.fori_loop(0, _ROT // 16, body, (zi, zi, zi))

    for q in range(_NVREG):
        li[pl.ds(q * 16, 16)] = zi
        lg[pl.ds(q * 16, 16)] = zf

    for i in range(3):
        plsc.store_scatter(li, [ranks[i]], t_idx[i])
        plsc.store_scatter(lg, [ranks[i]], t_val[i])

    pltpu.sync_copy(li, shi.at[pl.ds(sid * _C, _C)])
    pltpu.sync_copy(lg, shg.at[pl.ds(sid * _C, _C)])

    plsc.subcore_barrier()

    @pl.when(chunk == 0)
    def _writeback():
        for q in range(_NVREG):
            li[pl.ds(q * 16, 16)] = zi
            lg[pl.ds(q * 16, 16)] = zf
        for t in range(8):
            slot = (row * 8 + t) * _C
            pltpu.sync_copy(shi.at[pl.ds(slot, _C)], mi)
            pltpu.sync_copy(shg.at[pl.ds(slot, _C)], mg)
            for q in range(_NVREG):
                sl = pl.ds(q * 16, 16)
                li[sl] = li[sl] + mi[sl]
                lg[sl] = lg[sl] + mg[sl]
        pltpu.sync_copy(li.at[pl.ds(0, _HALF)], idx_hbm.at[bb])
        pltpu.sync_copy(lg.at[pl.ds(0, _HALF)], yh_hbm.at[bb])


@functools.lru_cache(maxsize=1)
def _select():
    return functools.partial(
        pl.kernel,
        out_type=(
            jax.ShapeDtypeStruct((_B, _HALF), jnp.int32),
            jax.ShapeDtypeStruct((_B, _HALF), jnp.float32),
        ),
        mesh=plsc.VectorSubcoreMesh(
            core_axis_name="c", subcore_axis_name="s", num_cores=2, num_subcores=16
        ),
        compiler_params=pltpu.CompilerParams(
            needs_layout_passes=False, use_tc_tiling_on_sc=False
        ),
        scratch_types=[
            pltpu.VMEM((_ROT,), jnp.float32),
            pltpu.VMEM((_ROT,), jnp.int32),
            pltpu.VMEM((_C,), jnp.int32),
            pltpu.VMEM((_C,), jnp.float32),
            pltpu.VMEM((_C,), jnp.int32),
            pltpu.VMEM((_C,), jnp.float32),
            pltpu.VMEM_SHARED((16 * _C,), jnp.int32),
            pltpu.VMEM_SHARED((16 * _C,), jnp.float32),
        ],
    )(_select_kernel)


_NP = 24


def _gather_scale_body(idx_ref, yh_ref, *refs):
    b = pl.program_id(0)
    k = pl.program_id(1)
    xs, o_ref = refs[:_NP], refs[_NP]
    for j in range(_NP):
        o_ref[0, j] = xs[j][0, 0] * yh_ref[b, k * _NP + j]


def _gather_scale(x, idx, y_half):
    B, C, H, W = x.shape

    def mk(j):
        return lambda b, k, idx, yh: (b, idx[b, k * _NP + j], 0, 0)

    grid_spec = pltpu.PrefetchScalarGridSpec(
        num_scalar_prefetch=2,
        grid=(B, _HALF // _NP),
        in_specs=[pl.BlockSpec((1, 1, H, W), mk(j)) for j in range(_NP)],
        out_specs=pl.BlockSpec(
            (1, _NP, H, W), lambda b, k, idx, yh: (b, k, 0, 0)
        ),
    )
    return pl.pallas_call(
        _gather_scale_body,
        grid_spec=grid_spec,
        out_shape=jax.ShapeDtypeStruct((B, _HALF, H, W), jnp.float32),
        compiler_params=pltpu.CompilerParams(
            dimension_semantics=("arbitrary", "arbitrary"),
        ),
    )(idx, y_half, *([x] * _NP))


def kernel(x, w1, w2):
    b, c, h, w = x.shape
    y = jnp.mean(x, axis=(2, 3))
    y = y @ w1.T
    y = jax.nn.relu(y)
    y = y @ w2.T
    y = jax.nn.sigmoid(y)
    y = lax.optimization_barrier(y)

    yv4 = y.reshape(b, _NVREG, 16)
    yrot = jnp.concatenate(
        [jnp.roll(yv4, -r, axis=2).reshape(b, 1, c) for r in range(16)], axis=1
    ).reshape(b, 16 * c)
    l16 = jnp.arange(16, dtype=jnp.int32)
    sidx = (
        jnp.arange(_NVREG, dtype=jnp.int32)[None, :, None] * 16
        + (l16[None, None, :] + l16[:, None, None]) % 16
    ).reshape(16 * c)

    idx, y_half = _select()(yrot, sidx)

    return _gather_scale(x, idx, y_half)


# speedup vs baseline: 2.5709x; 1.0006x over previous
import functools

import jax
import jax.numpy as jnp
from jax import lax
from jax.experimental import pallas as pl
from jax.experimental.pallas import tpu as pltpu
from jax.experimental.pallas import tpu_sc as plsc

_B, _C = 4, 384
_HALF = _C // 2
_NVREG = _C // 16
_ROT = _C * 16


def _select_kernel(yrot_hbm, sidx_hbm, idx_hbm, yh_hbm, yv, sv, li, lg, mi, mg, shi, shg):
    cid = lax.axis_index("c")
    sid = lax.axis_index("s")
    wid = cid * 16 + sid
    bb = wid // 8
    row = bb % 2
    chunk = wid % 8

    pltpu.sync_copy(yrot_hbm.at[bb], yv)
    pltpu.sync_copy(sidx_hbm, sv)

    iota = lax.iota(jnp.int32, 16)
    ones = jnp.ones((16,), jnp.int32)
    zi = jnp.zeros((16,), jnp.int32)
    zf = jnp.zeros((16,), jnp.float32)

    t_val = [yv[pl.ds((chunk * 3 + i) * 16, 16)] for i in range(3)]
    t_idx = [iota + (chunk * 48 + i * 16) for i in range(3)]

    def body(it, cnts):
        off = it * 16
        srot = yv[pl.ds(off, 16)]
        si = sv[pl.ds(off, 16)]
        out = []
        for i in range(3):
            beats = (srot > t_val[i]) | ((srot == t_val[i]) & (si < t_idx[i]))
            out.append(cnts[i] + jnp.where(beats, ones, zi))
        return tuple(out)

    ranks = lax.fori_loop(0, _ROT // 16, body, (zi, zi, zi))

    for q in range(_NVREG):
        li[pl.ds(q * 16, 16)] = zi
        lg[pl.ds(q * 16, 16)] = zf

    for i in range(3):
        plsc.store_scatter(li, [ranks[i]], t_idx[i])
        plsc.store_scatter(lg, [ranks[i]], t_val[i])

    pltpu.sync_copy(li, shi.at[pl.ds(sid * _C, _C)])
    pltpu.sync_copy(lg, shg.at[pl.ds(sid * _C, _C)])

    plsc.subcore_barrier()

    @pl.when(chunk == 0)
    def _writeback():
        for q in range(_NVREG):
            li[pl.ds(q * 16, 16)] = zi
            lg[pl.ds(q * 16, 16)] = zf
        for t in range(8):
            slot = (row * 8 + t) * _C
            pltpu.sync_copy(shi.at[pl.ds(slot, _C)], mi)
            pltpu.sync_copy(shg.at[pl.ds(slot, _C)], mg)
            for q in range(_NVREG):
                sl = pl.ds(q * 16, 16)
                li[sl] = li[sl] + mi[sl]
                lg[sl] = lg[sl] + mg[sl]
        pltpu.sync_copy(li.at[pl.ds(0, _HALF)], idx_hbm.at[bb])
        pltpu.sync_copy(lg.at[pl.ds(0, _HALF)], yh_hbm.at[bb])


@functools.lru_cache(maxsize=1)
def _select():
    return functools.partial(
        pl.kernel,
        out_type=(
            jax.ShapeDtypeStruct((_B, _HALF), jnp.int32),
            jax.ShapeDtypeStruct((_B, _HALF), jnp.float32),
        ),
        mesh=plsc.VectorSubcoreMesh(
            core_axis_name="c", subcore_axis_name="s", num_cores=2, num_subcores=16
        ),
        compiler_params=pltpu.CompilerParams(
            needs_layout_passes=False, use_tc_tiling_on_sc=False
        ),
        scratch_types=[
            pltpu.VMEM((_ROT,), jnp.float32),
            pltpu.VMEM((_ROT,), jnp.int32),
            pltpu.VMEM((_C,), jnp.int32),
            pltpu.VMEM((_C,), jnp.float32),
            pltpu.VMEM((_C,), jnp.int32),
            pltpu.VMEM((_C,), jnp.float32),
            pltpu.VMEM_SHARED((16 * _C,), jnp.int32),
            pltpu.VMEM_SHARED((16 * _C,), jnp.float32),
        ],
    )(_select_kernel)


_NP = 32


def _gather_scale_body(idx_ref, yh_ref, *refs):
    b = pl.program_id(0)
    k = pl.program_id(1)
    xs, o_ref = refs[:_NP], refs[_NP]
    for j in range(_NP):
        o_ref[0, j] = xs[j][0, 0] * yh_ref[b, k * _NP + j]


def _gather_scale(x, idx, y_half):
    B, C, H, W = x.shape

    def mk(j):
        return lambda b, k, idx, yh: (b, idx[b, k * _NP + j], 0, 0)

    grid_spec = pltpu.PrefetchScalarGridSpec(
        num_scalar_prefetch=2,
        grid=(B, _HALF // _NP),
        in_specs=[pl.BlockSpec((1, 1, H, W), mk(j)) for j in range(_NP)],
        out_specs=pl.BlockSpec(
            (1, _NP, H, W), lambda b, k, idx, yh: (b, k, 0, 0)
        ),
    )
    return pl.pallas_call(
        _gather_scale_body,
        grid_spec=grid_spec,
        out_shape=jax.ShapeDtypeStruct((B, _HALF, H, W), jnp.float32),
        compiler_params=pltpu.CompilerParams(
            dimension_semantics=("arbitrary", "arbitrary"),
        ),
    )(idx, y_half, *([x] * _NP))


def kernel(x, w1, w2):
    b, c, h, w = x.shape
    y = jnp.mean(x, axis=(2, 3))
    y = y @ w1.T
    y = jax.nn.relu(y)
    y = y @ w2.T
    y = jax.nn.sigmoid(y)
    y = lax.optimization_barrier(y)

    yv4 = y.reshape(b, _NVREG, 16)
    yrot = jnp.concatenate(
        [jnp.roll(yv4, -r, axis=2).reshape(b, 1, c) for r in range(16)], axis=1
    ).reshape(b, 16 * c)
    l16 = jnp.arange(16, dtype=jnp.int32)
    sidx = (
        jnp.arange(_NVREG, dtype=jnp.int32)[None, :, None] * 16
        + (l16[None, None, :] + l16[:, None, None]) % 16
    ).reshape(16 * c)

    idx, y_half = _select()(yrot, sidx)

    return _gather_scale(x, idx, y_half)
